# Initial kernel scaffold; baseline (speedup 1.0000x reference)
import jax
import jax.numpy as jnp
from jax.experimental import pallas as pl


def kernel(xn, xe, edge_i, edge_j, K1Nopen, K2Nopen, K1Eopen, K2Eopen, KNclose, KE1, KE2, KN1, KN2, lin1_w, lin1_b, lin2_w, lin2_b):
    raise NotImplementedError("write your pallas kernel here")


# speedup vs baseline: 2.1955x; 2.1955x over previous
import jax
import jax.numpy as jnp
from jax import lax
from jax.experimental import pallas as pl
from jax.experimental.pallas import tpu as pltpu
from jax.experimental.pallas import tpu_sc as plsc

N = 10000
E = 320000
C = 64
CP = 128
H = 0.1

NC = 2
NS = 16
NW = NC * NS
PER_W = E // NW
CH = 80
NSTEP = PER_W // CH
ROWS_W = 624
TAIL_W = N - NS * ROWS_W

TLE = 2000
TLN = 2000


def _sc_gather_body(xn_hbm, ei_hbm, ej_hbm, gi_hbm, gj_hbm, idx_v, rows_v, sem):
  c = lax.axis_index("c")
  s = lax.axis_index("s")
  wid = s * NC + c
  base0 = wid * PER_W

  def step(k, carry):
    base = base0 + k * CH
    pltpu.sync_copy(ei_hbm.at[pl.ds(base, CH)], idx_v)
    pltpu.async_copy(xn_hbm.at[idx_v], rows_v, sem).wait()
    pltpu.sync_copy(rows_v, gi_hbm.at[pl.ds(base, CH)])
    pltpu.sync_copy(ej_hbm.at[pl.ds(base, CH)], idx_v)
    pltpu.async_copy(xn_hbm.at[idx_v], rows_v, sem).wait()
    pltpu.sync_copy(rows_v, gj_hbm.at[pl.ds(base, CH)])
    return carry

  lax.fori_loop(0, NSTEP, step, 0)


def _sc_gather(xn_rows, ei, ej):
  return pl.kernel(
      _sc_gather_body,
      out_type=(
          jax.ShapeDtypeStruct((E, CP), jnp.float32),
          jax.ShapeDtypeStruct((E, CP), jnp.float32),
      ),
      mesh=plsc.VectorSubcoreMesh(core_axis_name="c", subcore_axis_name="s"),
      scratch_types=[
          pltpu.VMEM((CH,), jnp.int32),
          pltpu.VMEM((CH, CP), jnp.float32),
          pltpu.SemaphoreType.DMA,
      ],
  )(xn_rows, ei, ej)


def _sc_scatter_body(xe_hbm, ei_hbm, ej_hbm, zeros_hbm, si_hbm, sj_hbm,
                     idx_v, rows_v, acc):
  c = lax.axis_index("c")
  s = lax.axis_index("s")
  wid = s * NC + c
  r0 = s * ROWS_W
  rt = NS * ROWS_W
  base0 = wid * PER_W

  def phase(idx_hbm, out_hbm):
    pltpu.sync_copy(zeros_hbm.at[pl.ds(r0, ROWS_W)], acc.at[pl.ds(r0, ROWS_W)])

    @pl.when(s == NS - 1)
    def _():
      pltpu.sync_copy(zeros_hbm.at[pl.ds(rt, TAIL_W)],
                      acc.at[pl.ds(rt, TAIL_W)])

    plsc.subcore_barrier()

    def step(k, carry):
      base = base0 + k * CH
      pltpu.sync_copy(xe_hbm.at[pl.ds(base, CH)], rows_v)
      pltpu.sync_copy(idx_hbm.at[pl.ds(base, CH)], idx_v)
      pltpu.sync_copy(rows_v, acc.at[idx_v], add=True)
      return carry

    lax.fori_loop(0, NSTEP, step, 0)
    plsc.subcore_barrier()
    pltpu.sync_copy(acc.at[pl.ds(r0, ROWS_W)], out_hbm.at[c, pl.ds(r0, ROWS_W)])

    @pl.when(s == NS - 1)
    def _():
      pltpu.sync_copy(acc.at[pl.ds(rt, TAIL_W)],
                      out_hbm.at[c, pl.ds(rt, TAIL_W)])

  phase(ei_hbm, si_hbm)
  plsc.subcore_barrier()
  phase(ej_hbm, sj_hbm)


def _sc_scatter(xe_rows, ei, ej, zeros_rows):
  return pl.kernel(
      _sc_scatter_body,
      out_type=(
          jax.ShapeDtypeStruct((NC, N, CP), jnp.float32),
          jax.ShapeDtypeStruct((NC, N, CP), jnp.float32),
      ),
      mesh=plsc.VectorSubcoreMesh(core_axis_name="c", subcore_axis_name="s"),
      scratch_types=[
          pltpu.VMEM((CH,), jnp.int32),
          pltpu.VMEM((CH, CP), jnp.float32),
          pltpu.VMEM_SHARED((N, CP), jnp.float32),
      ],
  )(xe_rows, ei, ej, zeros_rows)


def _stats(y):
  s1 = jnp.sum(y)
  s2 = jnp.sum(y * y)
  col = lax.broadcasted_iota(jnp.int32, (1, 8, 128), 2)
  row = lax.broadcasted_iota(jnp.int32, (1, 8, 128), 1)
  vals = jnp.where(col == 0, s1, jnp.where(col == 1, s2, 0.0))
  return jnp.where(row == 0, vals, 0.0)


def _pad(y):
  return jnp.concatenate([y, jnp.zeros_like(y)], axis=1)


def _mm1_stats_body(x_ref, w_ref, y_ref, p_ref):
  y = jnp.dot(x_ref[...], w_ref[...], preferred_element_type=jnp.float32)
  y_ref[...] = y
  p_ref[...] = _stats(y)


def _mm1_stats(x, w, tl):
  rows, cin = x.shape
  cout = w.shape[1]
  g = rows // tl
  return pl.pallas_call(
      _mm1_stats_body,
      grid=(g,),
      in_specs=[
          pl.BlockSpec((tl, cin), lambda i: (i, 0)),
          pl.BlockSpec((cin, cout), lambda i: (0, 0)),
      ],
      out_specs=[
          pl.BlockSpec((tl, cout), lambda i: (i, 0)),
          pl.BlockSpec((1, 8, 128), lambda i: (i, 0, 0)),
      ],
      out_shape=[
          jax.ShapeDtypeStruct((rows, cout), jnp.float32),
          jax.ShapeDtypeStruct((g, 8, 128), jnp.float32),
      ],
  )(x, w)


def _mm2_stats_body(x1_ref, x2_ref, w1_ref, w2_ref, y_ref, p_ref):
  y = jnp.dot(x1_ref[:, :C], w1_ref[...], preferred_element_type=jnp.float32)
  y += jnp.dot(x2_ref[:, :C], w2_ref[...], preferred_element_type=jnp.float32)
  y_ref[...] = y
  p_ref[...] = _stats(y)


def _mm2_stats(x1, x2, w1, w2, tl):
  rows = x1.shape[0]
  g = rows // tl
  return pl.pallas_call(
      _mm2_stats_body,
      grid=(g,),
      in_specs=[
          pl.BlockSpec((tl, CP), lambda i: (i, 0)),
          pl.BlockSpec((tl, CP), lambda i: (i, 0)),
          pl.BlockSpec((C, C), lambda i: (0, 0)),
          pl.BlockSpec((C, C), lambda i: (0, 0)),
      ],
      out_specs=[
          pl.BlockSpec((tl, C), lambda i: (i, 0)),
          pl.BlockSpec((1, 8, 128), lambda i: (i, 0, 0)),
      ],
      out_shape=[
          jax.ShapeDtypeStruct((rows, C), jnp.float32),
          jax.ShapeDtypeStruct((g, 8, 128), jnp.float32),
      ],
  )(x1, x2, w1, w2)


def _seg_mm2_stats_body(si_ref, sj_ref, w1_ref, w2_ref, y_ref, p_ref):
  si = si_ref[0, :, :C] + si_ref[1, :, :C]
  sj = sj_ref[0, :, :C] + sj_ref[1, :, :C]
  y = jnp.dot(si, w1_ref[...], preferred_element_type=jnp.float32)
  y += jnp.dot(sj, w2_ref[...], preferred_element_type=jnp.float32)
  y_ref[...] = y
  p_ref[...] = _stats(y)


def _seg_mm2_stats(si, sj, w1, w2, tl):
  rows = si.shape[1]
  g = rows // tl
  return pl.pallas_call(
      _seg_mm2_stats_body,
      grid=(g,),
      in_specs=[
          pl.BlockSpec((NC, tl, CP), lambda i: (0, i, 0)),
          pl.BlockSpec((NC, tl, CP), lambda i: (0, i, 0)),
          pl.BlockSpec((C, C), lambda i: (0, 0)),
          pl.BlockSpec((C, C), lambda i: (0, 0)),
      ],
      out_specs=[
          pl.BlockSpec((tl, C), lambda i: (i, 0)),
          pl.BlockSpec((1, 8, 128), lambda i: (i, 0, 0)),
      ],
      out_shape=[
          jax.ShapeDtypeStruct((rows, C), jnp.float32),
          jax.ShapeDtypeStruct((g, 8, 128), jnp.float32),
      ],
  )(si, sj, w1, w2)


def _ntm_body(y1_ref, w_ref, mv_ref, out_ref):
  m = mv_ref[0, 0]
  r = mv_ref[0, 1]
  z = jnp.tanh((y1_ref[...] - m) * r)
  out_ref[...] = jnp.dot(z, w_ref[...], preferred_element_type=jnp.float32)


def _ntm(y1, w, mv, tl):
  rows, cin = y1.shape
  cout = w.shape[1]
  g = rows // tl
  return pl.pallas_call(
      _ntm_body,
      grid=(g,),
      in_specs=[
          pl.BlockSpec((tl, cin), lambda i: (i, 0)),
          pl.BlockSpec((cin, cout), lambda i: (0, 0)),
          pl.BlockSpec((8, 128), lambda i: (0, 0)),
      ],
      out_specs=pl.BlockSpec((tl, cout), lambda i: (i, 0)),
      out_shape=jax.ShapeDtypeStruct((rows, cout), jnp.float32),
  )(y1, w, mv)


def _ntm_pad_body(y1_ref, w_ref, mv_ref, out_ref):
  m = mv_ref[0, 0]
  r = mv_ref[0, 1]
  z = jnp.tanh((y1_ref[...] - m) * r)
  out_ref[...] = _pad(
      jnp.dot(z, w_ref[...], preferred_element_type=jnp.float32))


def _ntm_pad(y1, w, mv, tl):
  rows, cin = y1.shape
  g = rows // tl
  return pl.pallas_call(
      _ntm_pad_body,
      grid=(g,),
      in_specs=[
          pl.BlockSpec((tl, cin), lambda i: (i, 0)),
          pl.BlockSpec((cin, C), lambda i: (0, 0)),
          pl.BlockSpec((8, 128), lambda i: (0, 0)),
      ],
      out_specs=pl.BlockSpec((tl, CP), lambda i: (i, 0)),
      out_shape=jax.ShapeDtypeStruct((rows, CP), jnp.float32),
  )(y1, w, mv)


def _ntm_stats_body(y1_ref, w_ref, mv_ref, out_ref, p_ref):
  m = mv_ref[0, 0]
  r = mv_ref[0, 1]
  z = jnp.tanh((y1_ref[...] - m) * r)
  y = jnp.dot(z, w_ref[...], preferred_element_type=jnp.float32)
  out_ref[...] = y
  p_ref[...] = _stats(y)


def _ntm_stats(y1, w, mv, tl):
  rows = y1.shape[0]
  g = rows // tl
  return pl.pallas_call(
      _ntm_stats_body,
      grid=(g,),
      in_specs=[
          pl.BlockSpec((tl, C), lambda i: (i, 0)),
          pl.BlockSpec((C, C), lambda i: (0, 0)),
          pl.BlockSpec((8, 128), lambda i: (0, 0)),
      ],
      out_specs=[
          pl.BlockSpec((tl, C), lambda i: (i, 0)),
          pl.BlockSpec((1, 8, 128), lambda i: (i, 0, 0)),
      ],
      out_shape=[
          jax.ShapeDtypeStruct((rows, C), jnp.float32),
          jax.ShapeDtypeStruct((g, 8, 128), jnp.float32),
      ],
  )(y1, w, mv)


def _axpy_norm_body(xe_ref, y2_ref, mv_ref, out_ref):
  m = mv_ref[0, 0]
  r = mv_ref[0, 1]
  upd = xe_ref[:, :C] + H * ((y2_ref[...] - m) * r)
  out_ref[...] = _pad(upd)


def _axpy_norm(xe, y2, mv, tl):
  rows = xe.shape[0]
  g = rows // tl
  return pl.pallas_call(
      _axpy_norm_body,
      grid=(g,),
      in_specs=[
          pl.BlockSpec((tl, CP), lambda i: (i, 0)),
          pl.BlockSpec((tl, C), lambda i: (i, 0)),
          pl.BlockSpec((8, 128), lambda i: (0, 0)),
      ],
      out_specs=pl.BlockSpec((tl, CP), lambda i: (i, 0)),
      out_shape=jax.ShapeDtypeStruct((rows, CP), jnp.float32),
  )(xe, y2, mv)


def _node_update_body(xn_ref, t1_ref, w_ref, mv_ref, out_ref):
  m = mv_ref[0, 0]
  r = mv_ref[0, 1]
  z = jnp.tanh((t1_ref[...] - m) * r)
  upd = xn_ref[:, :C] + H * jnp.dot(
      z, w_ref[...], preferred_element_type=jnp.float32)
  out_ref[...] = _pad(upd)


def _node_update(xn, t1, w, mv, tl):
  rows = xn.shape[0]
  g = rows // tl
  return pl.pallas_call(
      _node_update_body,
      grid=(g,),
      in_specs=[
          pl.BlockSpec((tl, CP), lambda i: (i, 0)),
          pl.BlockSpec((tl, C), lambda i: (i, 0)),
          pl.BlockSpec((C, C), lambda i: (0, 0)),
          pl.BlockSpec((8, 128), lambda i: (0, 0)),
      ],
      out_specs=pl.BlockSpec((tl, CP), lambda i: (i, 0)),
      out_shape=jax.ShapeDtypeStruct((rows, CP), jnp.float32),
  )(xn, t1, w, mv)


def _head_body(x_ref, kc_ref, w1_ref, b1_ref, w2_ref, b2_ref, o_ref):
  cvals = jnp.dot(x_ref[:, :C], kc_ref[...], preferred_element_type=jnp.float32)
  h = jnp.dot(cvals, w1_ref[...], preferred_element_type=jnp.float32)
  h += b1_ref[...]
  h = jnp.where(h > 0, h, jnp.exp(h) - 1.0)
  o = jnp.dot(h, w2_ref[...], preferred_element_type=jnp.float32)
  o_ref[...] = o + b2_ref[...]


def _head(x, kc, w1, b1, w2, b2, tl):
  rows = x.shape[0]
  g = rows // tl
  return pl.pallas_call(
      _head_body,
      grid=(g,),
      in_specs=[
          pl.BlockSpec((tl, CP), lambda i: (i, 0)),
          pl.BlockSpec((C, C), lambda i: (0, 0)),
          pl.BlockSpec((C, 256), lambda i: (0, 0)),
          pl.BlockSpec((1, 256), lambda i: (0, 0)),
          pl.BlockSpec((256, 1024), lambda i: (0, 0)),
          pl.BlockSpec((1, 1024), lambda i: (0, 0)),
      ],
      out_specs=pl.BlockSpec((tl, 1024), lambda i: (i, 0)),
      out_shape=jax.ShapeDtypeStruct((rows, 1024), jnp.float32),
  )(x, kc, w1, b1, w2, b2)


def _mv(p, count):
  s1 = jnp.sum(p[:, 0, 0])
  s2 = jnp.sum(p[:, 0, 1])
  m = s1 / count
  v = s2 / count - m * m
  r = lax.rsqrt(v + 1e-5)
  return jnp.zeros((8, 128), jnp.float32).at[0, 0].set(m).at[0, 1].set(r)


def kernel(xn, xe, edge_i, edge_j, K1Nopen, K2Nopen, K1Eopen, K2Eopen,
           KNclose, KE1, KE2, KN1, KN2, lin1_w, lin1_b, lin2_w, lin2_b):
  xn0 = xn[0].T
  xe0 = xe[0].T

  y, p = _mm1_stats(xn0, K1Nopen.T, TLN)
  xnr = _ntm_pad(y, K2Nopen.T, _mv(p, N * C), TLN)
  y, p = _mm1_stats(xe0, K1Eopen.T, TLE)
  xer = _ntm_pad(y, K2Eopen.T, _mv(p, E * C), TLE)

  zeros_rows = jnp.zeros((N, CP), jnp.float32)

  for i in range(KE1.shape[0]):
    ke1a, ke1b = KE1[i][:, :C], KE1[i][:, C:]
    wi = (0.5 * ke1a + ke1b).T
    wj = (0.5 * ke1a - ke1b).T
    kn1a, kn1b = KN1[i][:, :C], KN1[i][:, C:]
    vi = (0.5 * kn1a + kn1b).T
    vj = (0.5 * kn1a - kn1b).T

    gi, gj = _sc_gather(xnr, edge_i, edge_j)
    y1, p1 = _mm2_stats(gi, gj, wi, wj, TLE)
    y2, p2 = _ntm_stats(y1, KE2[i].T, _mv(p1, E * C), TLE)
    xer = _axpy_norm(xer, y2, _mv(p2, E * C), TLE)

    si, sj = _sc_scatter(xer, edge_i, edge_j, zeros_rows)
    t1, p3 = _seg_mm2_stats(si, sj, vi, vj, TLN)
    xnr = _node_update(xnr, t1, KN2[i].T, _mv(p3, N * C), TLN)

  return _head(xnr, KNclose.T, lin1_w.T, lin1_b[None], lin2_w.T, lin2_b[None],
               TLN)


# speedup vs baseline: 2.8788x; 1.3112x over previous
import jax
import jax.numpy as jnp
from jax import lax
from jax.experimental import pallas as pl
from jax.experimental.pallas import tpu as pltpu
from jax.experimental.pallas import tpu_sc as plsc

N = 10000
E = 320000
C = 64
CP = 128
H = 0.1

NC = 2
NS = 16
CH = 80
PER_SW = E // NS
STEPS = PER_SW // CH
NPAIR = STEPS // 2
ROWS_W = 624
TAIL_W = N - NS * ROWS_W

TLE = 2000
TLN = 2000


def _sc_gather_body(xn_hbm, ei3_hbm, ej3_hbm, gi_hbm, gj_hbm,
                    idx_v, bufs, gsems, ssems):
  c = lax.axis_index("c")
  s = lax.axis_index("s")

  def side(idx3_hbm, out_hbm):
    pltpu.sync_copy(idx3_hbm.at[s], idx_v)
    base0 = s * PER_SW

    def run(kbase, width):
      ds_ = [pltpu.async_copy(xn_hbm.at[idx_v.at[kbase + u]], bufs[u],
                              gsems[u]) for u in range(width)]
      sts = []
      for u in range(width):
        ds_[u].wait()
        sts.append(pltpu.async_copy(
            bufs[u], out_hbm.at[pl.ds(base0 + (kbase + u) * CH, CH)],
            ssems[u]))
      for st in sts:
        st.wait()

    def quad(q, carry):
      run(4 * q, 4)
      return carry

    lax.fori_loop(0, STEPS // 4, quad, 0)
    run((STEPS // 4) * 4, STEPS % 4)

  @pl.when(c == 0)
  def _():
    side(ei3_hbm, gi_hbm)

  @pl.when(c == 1)
  def _():
    side(ej3_hbm, gj_hbm)


def _sc_gather(xn_rows, ei3, ej3):
  return pl.kernel(
      _sc_gather_body,
      out_type=(
          jax.ShapeDtypeStruct((E, CP), jnp.float32),
          jax.ShapeDtypeStruct((E, CP), jnp.float32),
      ),
      mesh=plsc.VectorSubcoreMesh(core_axis_name="c", subcore_axis_name="s"),
      scratch_types=[
          pltpu.VMEM((STEPS, CH), jnp.int32),
          [pltpu.VMEM((CH, CP), jnp.float32)] * 4,
          [pltpu.SemaphoreType.DMA] * 4,
          [pltpu.SemaphoreType.DMA] * 4,
      ],
  )(xn_rows, ei3, ej3)


def _sc_scatter_body(xe_hbm, ei3_hbm, ej3_hbm, zeros_hbm, si_hbm, sj_hbm,
                     idx_v, bufs, acc, lsems, asems):
  c = lax.axis_index("c")
  s = lax.axis_index("s")
  r0 = s * ROWS_W
  rt = NS * ROWS_W

  pltpu.sync_copy(zeros_hbm.at[pl.ds(r0, ROWS_W)], acc.at[pl.ds(r0, ROWS_W)])

  @pl.when(s == NS - 1)
  def _():
    pltpu.sync_copy(zeros_hbm.at[pl.ds(rt, TAIL_W)], acc.at[pl.ds(rt, TAIL_W)])

  plsc.subcore_barrier()

  def side(idx3_hbm, out_hbm):
    base0 = s * PER_SW

    def run(kbase, width, idxoff):
      ds_ = [pltpu.async_copy(xe_hbm.at[pl.ds(base0 + (kbase + u) * CH, CH)],
                              bufs[u], lsems[u]) for u in range(width)]
      adds = []
      for u in range(width):
        ds_[u].wait()
        adds.append(pltpu.async_copy(bufs[u], acc.at[idx_v.at[idxoff + u]],
                                     asems[u], add=True))
      for a in adds:
        a.wait()

    def octet(q, carry):
      k8 = 8 * q
      pltpu.sync_copy(idx3_hbm.at[s, pl.ds(k8, 8)], idx_v)
      run(k8, 4, 0)
      run(k8 + 4, 4, 4)
      return carry

    lax.fori_loop(0, STEPS // 8, octet, 0)
    kt = (STEPS // 8) * 8
    pltpu.sync_copy(idx3_hbm.at[s, pl.ds(kt, STEPS % 8)],
                    idx_v.at[pl.ds(0, STEPS % 8)])
    run(kt, STEPS % 8, 0)
    plsc.subcore_barrier()
    pltpu.sync_copy(acc.at[pl.ds(r0, ROWS_W)], out_hbm.at[pl.ds(r0, ROWS_W)])

    @pl.when(s == NS - 1)
    def _():
      pltpu.sync_copy(acc.at[pl.ds(rt, TAIL_W)], out_hbm.at[pl.ds(rt, TAIL_W)])

  @pl.when(c == 0)
  def _():
    side(ei3_hbm, si_hbm)

  @pl.when(c == 1)
  def _():
    side(ej3_hbm, sj_hbm)


def _sc_scatter(xe_rows, ei3, ej3, zeros_rows):
  return pl.kernel(
      _sc_scatter_body,
      out_type=(
          jax.ShapeDtypeStruct((N, CP), jnp.float32),
          jax.ShapeDtypeStruct((N, CP), jnp.float32),
      ),
      mesh=plsc.VectorSubcoreMesh(core_axis_name="c", subcore_axis_name="s"),
      scratch_types=[
          pltpu.VMEM((8, CH), jnp.int32),
          [pltpu.VMEM((CH, CP), jnp.float32)] * 4,
          pltpu.VMEM_SHARED((N, CP), jnp.float32),
          [pltpu.SemaphoreType.DMA] * 4,
          [pltpu.SemaphoreType.DMA] * 4,
      ],
  )(xe_rows, ei3, ej3, zeros_rows)


def _stats(y):
  s1 = jnp.sum(y)
  s2 = jnp.sum(y * y)
  col = lax.broadcasted_iota(jnp.int32, (1, 8, 128), 2)
  row = lax.broadcasted_iota(jnp.int32, (1, 8, 128), 1)
  vals = jnp.where(col == 0, s1, jnp.where(col == 1, s2, 0.0))
  return jnp.where(row == 0, vals, 0.0)


def _pad(y):
  return jnp.concatenate([y, jnp.zeros_like(y)], axis=1)


def _mm1_stats_body(x_ref, w_ref, y_ref, p_ref):
  y = jnp.dot(x_ref[...], w_ref[...], preferred_element_type=jnp.float32)
  y_ref[...] = y
  p_ref[...] = _stats(y)


def _mm1_stats(x, w, tl):
  rows, cin = x.shape
  cout = w.shape[1]
  g = rows // tl
  return pl.pallas_call(
      _mm1_stats_body,
      grid=(g,),
      in_specs=[
          pl.BlockSpec((tl, cin), lambda i: (i, 0)),
          pl.BlockSpec((cin, cout), lambda i: (0, 0)),
      ],
      out_specs=[
          pl.BlockSpec((tl, cout), lambda i: (i, 0)),
          pl.BlockSpec((1, 8, 128), lambda i: (i, 0, 0)),
      ],
      out_shape=[
          jax.ShapeDtypeStruct((rows, cout), jnp.float32),
          jax.ShapeDtypeStruct((g, 8, 128), jnp.float32),
      ],
  )(x, w)


def _mm2_stats_body(x1_ref, x2_ref, w1_ref, w2_ref, y_ref, p_ref):
  y = jnp.dot(x1_ref[:, :C], w1_ref[...], preferred_element_type=jnp.float32)
  y += jnp.dot(x2_ref[:, :C], w2_ref[...], preferred_element_type=jnp.float32)
  y_ref[...] = y
  p_ref[...] = _stats(y)


def _mm2_stats(x1, x2, w1, w2, tl):
  rows = x1.shape[0]
  g = rows // tl
  return pl.pallas_call(
      _mm2_stats_body,
      grid=(g,),
      in_specs=[
          pl.BlockSpec((tl, CP), lambda i: (i, 0)),
          pl.BlockSpec((tl, CP), lambda i: (i, 0)),
          pl.BlockSpec((C, C), lambda i: (0, 0)),
          pl.BlockSpec((C, C), lambda i: (0, 0)),
      ],
      out_specs=[
          pl.BlockSpec((tl, C), lambda i: (i, 0)),
          pl.BlockSpec((1, 8, 128), lambda i: (i, 0, 0)),
      ],
      out_shape=[
          jax.ShapeDtypeStruct((rows, C), jnp.float32),
          jax.ShapeDtypeStruct((g, 8, 128), jnp.float32),
      ],
  )(x1, x2, w1, w2)


def _seg_mm2_stats_body(si_ref, sj_ref, w1_ref, w2_ref, y_ref, p_ref):
  y = jnp.dot(si_ref[:, :C], w1_ref[...], preferred_element_type=jnp.float32)
  y += jnp.dot(sj_ref[:, :C], w2_ref[...], preferred_element_type=jnp.float32)
  y_ref[...] = y
  p_ref[...] = _stats(y)


def _seg_mm2_stats(si, sj, w1, w2, tl):
  rows = si.shape[0]
  g = rows // tl
  return pl.pallas_call(
      _seg_mm2_stats_body,
      grid=(g,),
      in_specs=[
          pl.BlockSpec((tl, CP), lambda i: (i, 0)),
          pl.BlockSpec((tl, CP), lambda i: (i, 0)),
          pl.BlockSpec((C, C), lambda i: (0, 0)),
          pl.BlockSpec((C, C), lambda i: (0, 0)),
      ],
      out_specs=[
          pl.BlockSpec((tl, C), lambda i: (i, 0)),
          pl.BlockSpec((1, 8, 128), lambda i: (i, 0, 0)),
      ],
      out_shape=[
          jax.ShapeDtypeStruct((rows, C), jnp.float32),
          jax.ShapeDtypeStruct((g, 8, 128), jnp.float32),
      ],
  )(si, sj, w1, w2)


def _ntm_body(y1_ref, w_ref, mv_ref, out_ref):
  m = mv_ref[0, 0]
  r = mv_ref[0, 1]
  z = jnp.tanh((y1_ref[...] - m) * r)
  out_ref[...] = jnp.dot(z, w_ref[...], preferred_element_type=jnp.float32)


def _ntm(y1, w, mv, tl):
  rows, cin = y1.shape
  cout = w.shape[1]
  g = rows // tl
  return pl.pallas_call(
      _ntm_body,
      grid=(g,),
      in_specs=[
          pl.BlockSpec((tl, cin), lambda i: (i, 0)),
          pl.BlockSpec((cin, cout), lambda i: (0, 0)),
          pl.BlockSpec((8, 128), lambda i: (0, 0)),
      ],
      out_specs=pl.BlockSpec((tl, cout), lambda i: (i, 0)),
      out_shape=jax.ShapeDtypeStruct((rows, cout), jnp.float32),
  )(y1, w, mv)


def _ntm_pad_body(y1_ref, w_ref, mv_ref, out_ref):
  m = mv_ref[0, 0]
  r = mv_ref[0, 1]
  z = jnp.tanh((y1_ref[...] - m) * r)
  out_ref[...] = _pad(
      jnp.dot(z, w_ref[...], preferred_element_type=jnp.float32))


def _ntm_pad(y1, w, mv, tl):
  rows, cin = y1.shape
  g = rows // tl
  return pl.pallas_call(
      _ntm_pad_body,
      grid=(g,),
      in_specs=[
          pl.BlockSpec((tl, cin), lambda i: (i, 0)),
          pl.BlockSpec((cin, C), lambda i: (0, 0)),
          pl.BlockSpec((8, 128), lambda i: (0, 0)),
      ],
      out_specs=pl.BlockSpec((tl, CP), lambda i: (i, 0)),
      out_shape=jax.ShapeDtypeStruct((rows, CP), jnp.float32),
  )(y1, w, mv)


def _ntm_stats_body(y1_ref, w_ref, mv_ref, out_ref, p_ref):
  m = mv_ref[0, 0]
  r = mv_ref[0, 1]
  z = jnp.tanh((y1_ref[...] - m) * r)
  y = jnp.dot(z, w_ref[...], preferred_element_type=jnp.float32)
  out_ref[...] = y
  p_ref[...] = _stats(y)


def _ntm_stats(y1, w, mv, tl):
  rows = y1.shape[0]
  g = rows // tl
  return pl.pallas_call(
      _ntm_stats_body,
      grid=(g,),
      in_specs=[
          pl.BlockSpec((tl, C), lambda i: (i, 0)),
          pl.BlockSpec((C, C), lambda i: (0, 0)),
          pl.BlockSpec((8, 128), lambda i: (0, 0)),
      ],
      out_specs=[
          pl.BlockSpec((tl, C), lambda i: (i, 0)),
          pl.BlockSpec((1, 8, 128), lambda i: (i, 0, 0)),
      ],
      out_shape=[
          jax.ShapeDtypeStruct((rows, C), jnp.float32),
          jax.ShapeDtypeStruct((g, 8, 128), jnp.float32),
      ],
  )(y1, w, mv)


def _axpy_norm_body(xe_ref, y2_ref, mv_ref, out_ref):
  m = mv_ref[0, 0]
  r = mv_ref[0, 1]
  upd = xe_ref[:, :C] + H * ((y2_ref[...] - m) * r)
  out_ref[...] = _pad(upd)


def _axpy_norm(xe, y2, mv, tl):
  rows = xe.shape[0]
  g = rows // tl
  return pl.pallas_call(
      _axpy_norm_body,
      grid=(g,),
      in_specs=[
          pl.BlockSpec((tl, CP), lambda i: (i, 0)),
          pl.BlockSpec((tl, C), lambda i: (i, 0)),
          pl.BlockSpec((8, 128), lambda i: (0, 0)),
      ],
      out_specs=pl.BlockSpec((tl, CP), lambda i: (i, 0)),
      out_shape=jax.ShapeDtypeStruct((rows, CP), jnp.float32),
  )(xe, y2, mv)


def _node_update_body(xn_ref, t1_ref, w_ref, mv_ref, out_ref):
  m = mv_ref[0, 0]
  r = mv_ref[0, 1]
  z = jnp.tanh((t1_ref[...] - m) * r)
  upd = xn_ref[:, :C] + H * jnp.dot(
      z, w_ref[...], preferred_element_type=jnp.float32)
  out_ref[...] = _pad(upd)


def _node_update(xn, t1, w, mv, tl):
  rows = xn.shape[0]
  g = rows // tl
  return pl.pallas_call(
      _node_update_body,
      grid=(g,),
      in_specs=[
          pl.BlockSpec((tl, CP), lambda i: (i, 0)),
          pl.BlockSpec((tl, C), lambda i: (i, 0)),
          pl.BlockSpec((C, C), lambda i: (0, 0)),
          pl.BlockSpec((8, 128), lambda i: (0, 0)),
      ],
      out_specs=pl.BlockSpec((tl, CP), lambda i: (i, 0)),
      out_shape=jax.ShapeDtypeStruct((rows, CP), jnp.float32),
  )(xn, t1, w, mv)


def _head_body(x_ref, kc_ref, w1_ref, b1_ref, w2_ref, b2_ref, o_ref):
  cvals = jnp.dot(x_ref[:, :C], kc_ref[...], preferred_element_type=jnp.float32)
  h = jnp.dot(cvals, w1_ref[...], preferred_element_type=jnp.float32)
  h += b1_ref[...]
  h = jnp.where(h > 0, h, jnp.exp(h) - 1.0)
  o = jnp.dot(h, w2_ref[...], preferred_element_type=jnp.float32)
  o_ref[...] = o + b2_ref[...]


def _head(x, kc, w1, b1, w2, b2, tl):
  rows = x.shape[0]
  g = rows // tl
  return pl.pallas_call(
      _head_body,
      grid=(g,),
      in_specs=[
          pl.BlockSpec((tl, CP), lambda i: (i, 0)),
          pl.BlockSpec((C, C), lambda i: (0, 0)),
          pl.BlockSpec((C, 256), lambda i: (0, 0)),
          pl.BlockSpec((1, 256), lambda i: (0, 0)),
          pl.BlockSpec((256, 1024), lambda i: (0, 0)),
          pl.BlockSpec((1, 1024), lambda i: (0, 0)),
      ],
      out_specs=pl.BlockSpec((tl, 1024), lambda i: (i, 0)),
      out_shape=jax.ShapeDtypeStruct((rows, 1024), jnp.float32),
  )(x, kc, w1, b1, w2, b2)


def _mv(p, count):
  s1 = jnp.sum(p[:, 0, 0])
  s2 = jnp.sum(p[:, 0, 1])
  m = s1 / count
  v = s2 / count - m * m
  r = lax.rsqrt(v + 1e-5)
  return jnp.zeros((8, 128), jnp.float32).at[0, 0].set(m).at[0, 1].set(r)


def kernel(xn, xe, edge_i, edge_j, K1Nopen, K2Nopen, K1Eopen, K2Eopen,
           KNclose, KE1, KE2, KN1, KN2, lin1_w, lin1_b, lin2_w, lin2_b):
  xn0 = xn[0].T
  xe0 = xe[0].T

  y, p = _mm1_stats(xn0, K1Nopen.T, TLN)
  xnr = _ntm_pad(y, K2Nopen.T, _mv(p, N * C), TLN)
  y, p = _mm1_stats(xe0, K1Eopen.T, TLE)
  xer = _ntm_pad(y, K2Eopen.T, _mv(p, E * C), TLE)

  zeros_rows = jnp.zeros((N, CP), jnp.float32)
  ei3 = edge_i.reshape(NS, STEPS, CH)
  ej3 = edge_j.reshape(NS, STEPS, CH)

  for i in range(KE1.shape[0]):
    ke1a, ke1b = KE1[i][:, :C], KE1[i][:, C:]
    wi = (0.5 * ke1a + ke1b).T
    wj = (0.5 * ke1a - ke1b).T
    kn1a, kn1b = KN1[i][:, :C], KN1[i][:, C:]
    vi = (0.5 * kn1a + kn1b).T
    vj = (0.5 * kn1a - kn1b).T

    gi, gj = _sc_gather(xnr, ei3, ej3)
    y1, p1 = _mm2_stats(gi, gj, wi, wj, TLE)
    y2, p2 = _ntm_stats(y1, KE2[i].T, _mv(p1, E * C), TLE)
    xer = _axpy_norm(xer, y2, _mv(p2, E * C), TLE)

    si, sj = _sc_scatter(xer, ei3, ej3, zeros_rows)
    t1, p3 = _seg_mm2_stats(si, sj, vi, vj, TLN)
    xnr = _node_update(xnr, t1, KN2[i].T, _mv(p3, N * C), TLN)

  return _head(xnr, KNclose.T, lin1_w.T, lin1_b[None], lin2_w.T, lin2_b[None],
               TLN)


# speedup vs baseline: 3.3388x; 1.1598x over previous
import jax
import jax.numpy as jnp
from jax import lax
from jax.experimental import pallas as pl
from jax.experimental.pallas import tpu as pltpu
from jax.experimental.pallas import tpu_sc as plsc

N = 10000
E = 320000
C = 64
CP = 128
H = 0.1

NC = 2
NS = 16
CH = 80
PER_SW = E // NS
STEPS = PER_SW // CH
NPAIR = STEPS // 2
ROWS_W = 624
TAIL_W = N - NS * ROWS_W

TLE = 4000
TLN = 2000


def _sc_gather_body(xn_hbm, ei3_hbm, ej3_hbm, gi_hbm, gj_hbm,
                    idx_v, bufs, gsems, ssems):
  c = lax.axis_index("c")
  s = lax.axis_index("s")

  def side(idx3_hbm, out_hbm):
    pltpu.sync_copy(idx3_hbm.at[s], idx_v)
    base0 = s * PER_SW

    def run(kbase, width):
      ds_ = [pltpu.async_copy(xn_hbm.at[idx_v.at[kbase + u]], bufs[u],
                              gsems[u]) for u in range(width)]
      sts = []
      for u in range(width):
        ds_[u].wait()
        sts.append(pltpu.async_copy(
            bufs[u], out_hbm.at[pl.ds(base0 + (kbase + u) * CH, CH)],
            ssems[u]))
      for st in sts:
        st.wait()

    def quad(q, carry):
      run(4 * q, 4)
      return carry

    lax.fori_loop(0, STEPS // 4, quad, 0)
    run((STEPS // 4) * 4, STEPS % 4)

  @pl.when(c == 0)
  def _():
    side(ei3_hbm, gi_hbm)

  @pl.when(c == 1)
  def _():
    side(ej3_hbm, gj_hbm)


def _sc_gather(xn_rows, ei3, ej3):
  return pl.kernel(
      _sc_gather_body,
      out_type=(
          jax.ShapeDtypeStruct((E, CP), jnp.float32),
          jax.ShapeDtypeStruct((E, CP), jnp.float32),
      ),
      mesh=plsc.VectorSubcoreMesh(core_axis_name="c", subcore_axis_name="s"),
      scratch_types=[
          pltpu.VMEM((STEPS, CH), jnp.int32),
          [pltpu.VMEM((CH, CP), jnp.float32)] * 4,
          [pltpu.SemaphoreType.DMA] * 4,
          [pltpu.SemaphoreType.DMA] * 4,
      ],
  )(xn_rows, ei3, ej3)


def _sc_scatter_body(xe_hbm, ei3_hbm, ej3_hbm, zeros_hbm, si_hbm, sj_hbm,
                     idx_v, bufs, acc, lsems, asems):
  c = lax.axis_index("c")
  s = lax.axis_index("s")
  r0 = s * ROWS_W
  rt = NS * ROWS_W

  pltpu.sync_copy(zeros_hbm.at[pl.ds(r0, ROWS_W)], acc.at[pl.ds(r0, ROWS_W)])

  @pl.when(s == NS - 1)
  def _():
    pltpu.sync_copy(zeros_hbm.at[pl.ds(rt, TAIL_W)], acc.at[pl.ds(rt, TAIL_W)])

  plsc.subcore_barrier()

  def side(idx3_hbm, out_hbm):
    base0 = s * PER_SW

    def run(kbase, width, idxoff):
      ds_ = [pltpu.async_copy(xe_hbm.at[pl.ds(base0 + (kbase + u) * CH, CH)],
                              bufs[u], lsems[u]) for u in range(width)]
      adds = []
      for u in range(width):
        ds_[u].wait()
        adds.append(pltpu.async_copy(bufs[u], acc.at[idx_v.at[idxoff + u]],
                                     asems[u], add=True))
      for a in adds:
        a.wait()

    def octet(q, carry):
      k8 = 8 * q
      pltpu.sync_copy(idx3_hbm.at[s, pl.ds(k8, 8)], idx_v)
      run(k8, 4, 0)
      run(k8 + 4, 4, 4)
      return carry

    lax.fori_loop(0, STEPS // 8, octet, 0)
    kt = (STEPS // 8) * 8
    pltpu.sync_copy(idx3_hbm.at[s, pl.ds(kt, STEPS % 8)],
                    idx_v.at[pl.ds(0, STEPS % 8)])
    run(kt, STEPS % 8, 0)
    plsc.subcore_barrier()
    pltpu.sync_copy(acc.at[pl.ds(r0, ROWS_W)], out_hbm.at[pl.ds(r0, ROWS_W)])

    @pl.when(s == NS - 1)
    def _():
      pltpu.sync_copy(acc.at[pl.ds(rt, TAIL_W)], out_hbm.at[pl.ds(rt, TAIL_W)])

  @pl.when(c == 0)
  def _():
    side(ei3_hbm, si_hbm)

  @pl.when(c == 1)
  def _():
    side(ej3_hbm, sj_hbm)


def _sc_scatter(xe_rows, ei3, ej3, zeros_rows):
  return pl.kernel(
      _sc_scatter_body,
      out_type=(
          jax.ShapeDtypeStruct((N, CP), jnp.float32),
          jax.ShapeDtypeStruct((N, CP), jnp.float32),
      ),
      mesh=plsc.VectorSubcoreMesh(core_axis_name="c", subcore_axis_name="s"),
      scratch_types=[
          pltpu.VMEM((8, CH), jnp.int32),
          [pltpu.VMEM((CH, CP), jnp.float32)] * 4,
          pltpu.VMEM_SHARED((N, CP), jnp.float32),
          [pltpu.SemaphoreType.DMA] * 4,
          [pltpu.SemaphoreType.DMA] * 4,
      ],
  )(xe_rows, ei3, ej3, zeros_rows)


def _stats(y):
  s1 = jnp.sum(y)
  s2 = jnp.sum(y * y)
  col = lax.broadcasted_iota(jnp.int32, (1, 8, 128), 2)
  row = lax.broadcasted_iota(jnp.int32, (1, 8, 128), 1)
  vals = jnp.where(col == 0, s1, jnp.where(col == 1, s2, 0.0))
  return jnp.where(row == 0, vals, 0.0)


def _pad(y):
  return jnp.concatenate([y, jnp.zeros_like(y)], axis=1)


def _mm1_stats_body(x_ref, w_ref, y_ref, p_ref):
  y = jnp.dot(x_ref[...], w_ref[...], preferred_element_type=jnp.float32)
  y_ref[...] = y
  p_ref[...] = _stats(y)


def _mm1_stats(x, w, tl):
  rows, cin = x.shape
  cout = w.shape[1]
  g = rows // tl
  return pl.pallas_call(
      _mm1_stats_body,
      grid=(g,),
      in_specs=[
          pl.BlockSpec((tl, cin), lambda i: (i, 0)),
          pl.BlockSpec((cin, cout), lambda i: (0, 0)),
      ],
      out_specs=[
          pl.BlockSpec((tl, cout), lambda i: (i, 0)),
          pl.BlockSpec((1, 8, 128), lambda i: (i, 0, 0)),
      ],
      out_shape=[
          jax.ShapeDtypeStruct((rows, cout), jnp.float32),
          jax.ShapeDtypeStruct((g, 8, 128), jnp.float32),
      ],
  )(x, w)


def _mm2_stats_body(x1_ref, x2_ref, w1_ref, w2_ref, y_ref, p_ref):
  y = jnp.dot(x1_ref[:, :C], w1_ref[...], preferred_element_type=jnp.float32)
  y += jnp.dot(x2_ref[:, :C], w2_ref[...], preferred_element_type=jnp.float32)
  y_ref[...] = y
  p_ref[...] = _stats(y)


def _mm2_stats(x1, x2, w1, w2, tl):
  rows = x1.shape[0]
  g = rows // tl
  return pl.pallas_call(
      _mm2_stats_body,
      grid=(g,),
      in_specs=[
          pl.BlockSpec((tl, CP), lambda i: (i, 0)),
          pl.BlockSpec((tl, CP), lambda i: (i, 0)),
          pl.BlockSpec((C, C), lambda i: (0, 0)),
          pl.BlockSpec((C, C), lambda i: (0, 0)),
      ],
      out_specs=[
          pl.BlockSpec((tl, C), lambda i: (i, 0)),
          pl.BlockSpec((1, 8, 128), lambda i: (i, 0, 0)),
      ],
      out_shape=[
          jax.ShapeDtypeStruct((rows, C), jnp.float32),
          jax.ShapeDtypeStruct((g, 8, 128), jnp.float32),
      ],
  )(x1, x2, w1, w2)


def _seg_mm2_stats_body(si_ref, sj_ref, w1_ref, w2_ref, y_ref, p_ref):
  y = jnp.dot(si_ref[:, :C], w1_ref[...], preferred_element_type=jnp.float32)
  y += jnp.dot(sj_ref[:, :C], w2_ref[...], preferred_element_type=jnp.float32)
  y_ref[...] = y
  p_ref[...] = _stats(y)


def _seg_mm2_stats(si, sj, w1, w2, tl):
  rows = si.shape[0]
  g = rows // tl
  return pl.pallas_call(
      _seg_mm2_stats_body,
      grid=(g,),
      in_specs=[
          pl.BlockSpec((tl, CP), lambda i: (i, 0)),
          pl.BlockSpec((tl, CP), lambda i: (i, 0)),
          pl.BlockSpec((C, C), lambda i: (0, 0)),
          pl.BlockSpec((C, C), lambda i: (0, 0)),
      ],
      out_specs=[
          pl.BlockSpec((tl, C), lambda i: (i, 0)),
          pl.BlockSpec((1, 8, 128), lambda i: (i, 0, 0)),
      ],
      out_shape=[
          jax.ShapeDtypeStruct((rows, C), jnp.float32),
          jax.ShapeDtypeStruct((g, 8, 128), jnp.float32),
      ],
  )(si, sj, w1, w2)


def _ntm_body(y1_ref, w_ref, mv_ref, out_ref):
  m = mv_ref[0, 0]
  r = mv_ref[0, 1]
  z = jnp.tanh((y1_ref[...] - m) * r)
  out_ref[...] = jnp.dot(z, w_ref[...], preferred_element_type=jnp.float32)


def _ntm(y1, w, mv, tl):
  rows, cin = y1.shape
  cout = w.shape[1]
  g = rows // tl
  return pl.pallas_call(
      _ntm_body,
      grid=(g,),
      in_specs=[
          pl.BlockSpec((tl, cin), lambda i: (i, 0)),
          pl.BlockSpec((cin, cout), lambda i: (0, 0)),
          pl.BlockSpec((8, 128), lambda i: (0, 0)),
      ],
      out_specs=pl.BlockSpec((tl, cout), lambda i: (i, 0)),
      out_shape=jax.ShapeDtypeStruct((rows, cout), jnp.float32),
  )(y1, w, mv)


def _ntm_pad_body(y1_ref, w_ref, mv_ref, out_ref):
  m = mv_ref[0, 0]
  r = mv_ref[0, 1]
  z = jnp.tanh((y1_ref[...] - m) * r)
  out_ref[...] = _pad(
      jnp.dot(z, w_ref[...], preferred_element_type=jnp.float32))


def _ntm_pad(y1, w, mv, tl):
  rows, cin = y1.shape
  g = rows // tl
  return pl.pallas_call(
      _ntm_pad_body,
      grid=(g,),
      in_specs=[
          pl.BlockSpec((tl, cin), lambda i: (i, 0)),
          pl.BlockSpec((cin, C), lambda i: (0, 0)),
          pl.BlockSpec((8, 128), lambda i: (0, 0)),
      ],
      out_specs=pl.BlockSpec((tl, CP), lambda i: (i, 0)),
      out_shape=jax.ShapeDtypeStruct((rows, CP), jnp.float32),
  )(y1, w, mv)


def _ntm_stats_body(y1_ref, w_ref, mv_ref, out_ref, p_ref):
  m = mv_ref[0, 0]
  r = mv_ref[0, 1]
  z = jnp.tanh((y1_ref[...] - m) * r)
  y = jnp.dot(z, w_ref[...], preferred_element_type=jnp.float32)
  out_ref[...] = y
  p_ref[...] = _stats(y)


def _ntm_stats(y1, w, mv, tl):
  rows = y1.shape[0]
  g = rows // tl
  return pl.pallas_call(
      _ntm_stats_body,
      grid=(g,),
      in_specs=[
          pl.BlockSpec((tl, C), lambda i: (i, 0)),
          pl.BlockSpec((C, C), lambda i: (0, 0)),
          pl.BlockSpec((8, 128), lambda i: (0, 0)),
      ],
      out_specs=[
          pl.BlockSpec((tl, C), lambda i: (i, 0)),
          pl.BlockSpec((1, 8, 128), lambda i: (i, 0, 0)),
      ],
      out_shape=[
          jax.ShapeDtypeStruct((rows, C), jnp.float32),
          jax.ShapeDtypeStruct((g, 8, 128), jnp.float32),
      ],
  )(y1, w, mv)


def _axpy_norm_body(xe_ref, y2_ref, mv_ref, out_ref):
  m = mv_ref[0, 0]
  r = mv_ref[0, 1]
  upd = xe_ref[:, :C] + H * ((y2_ref[...] - m) * r)
  out_ref[...] = _pad(upd)


def _axpy_norm(xe, y2, mv, tl):
  rows = xe.shape[0]
  g = rows // tl
  return pl.pallas_call(
      _axpy_norm_body,
      grid=(g,),
      in_specs=[
          pl.BlockSpec((tl, CP), lambda i: (i, 0)),
          pl.BlockSpec((tl, C), lambda i: (i, 0)),
          pl.BlockSpec((8, 128), lambda i: (0, 0)),
      ],
      out_specs=pl.BlockSpec((tl, CP), lambda i: (i, 0)),
      out_shape=jax.ShapeDtypeStruct((rows, CP), jnp.float32),
  )(xe, y2, mv)


def _node_update_body(xn_ref, t1_ref, w_ref, mv_ref, out_ref):
  m = mv_ref[0, 0]
  r = mv_ref[0, 1]
  z = jnp.tanh((t1_ref[...] - m) * r)
  upd = xn_ref[:, :C] + H * jnp.dot(
      z, w_ref[...], preferred_element_type=jnp.float32)
  out_ref[...] = _pad(upd)


def _node_update(xn, t1, w, mv, tl):
  rows = xn.shape[0]
  g = rows // tl
  return pl.pallas_call(
      _node_update_body,
      grid=(g,),
      in_specs=[
          pl.BlockSpec((tl, CP), lambda i: (i, 0)),
          pl.BlockSpec((tl, C), lambda i: (i, 0)),
          pl.BlockSpec((C, C), lambda i: (0, 0)),
          pl.BlockSpec((8, 128), lambda i: (0, 0)),
      ],
      out_specs=pl.BlockSpec((tl, CP), lambda i: (i, 0)),
      out_shape=jax.ShapeDtypeStruct((rows, CP), jnp.float32),
  )(xn, t1, w, mv)


def _head_body(x_ref, kc_ref, w1_ref, b1_ref, w2_ref, b2_ref, o_ref):
  cvals = jnp.dot(x_ref[:, :C], kc_ref[...], preferred_element_type=jnp.float32)
  h = jnp.dot(cvals, w1_ref[...], preferred_element_type=jnp.float32)
  h += b1_ref[...]
  h = jnp.where(h > 0, h, jnp.exp(h) - 1.0)
  o = jnp.dot(h, w2_ref[...], preferred_element_type=jnp.float32)
  o_ref[...] = o + b2_ref[...]


def _head(x, kc, w1, b1, w2, b2, tl):
  rows = x.shape[0]
  g = rows // tl
  return pl.pallas_call(
      _head_body,
      grid=(g,),
      in_specs=[
          pl.BlockSpec((tl, CP), lambda i: (i, 0)),
          pl.BlockSpec((C, C), lambda i: (0, 0)),
          pl.BlockSpec((C, 256), lambda i: (0, 0)),
          pl.BlockSpec((1, 256), lambda i: (0, 0)),
          pl.BlockSpec((256, 1024), lambda i: (0, 0)),
          pl.BlockSpec((1, 1024), lambda i: (0, 0)),
      ],
      out_specs=pl.BlockSpec((tl, 1024), lambda i: (i, 0)),
      out_shape=jax.ShapeDtypeStruct((rows, 1024), jnp.float32),
  )(x, kc, w1, b1, w2, b2)


def _mv(p, count):
  s1 = jnp.sum(p[:, 0, 0])
  s2 = jnp.sum(p[:, 0, 1])
  m = s1 / count
  v = s2 / count - m * m
  r = lax.rsqrt(v + 1e-5)
  return jnp.zeros((8, 128), jnp.float32).at[0, 0].set(m).at[0, 1].set(r)


def kernel(xn, xe, edge_i, edge_j, K1Nopen, K2Nopen, K1Eopen, K2Eopen,
           KNclose, KE1, KE2, KN1, KN2, lin1_w, lin1_b, lin2_w, lin2_b):
  xn0 = xn[0].T
  xe0 = xe[0].T

  y, p = _mm1_stats(xn0, K1Nopen.T, TLN)
  xnr = _ntm_pad(y, K2Nopen.T, _mv(p, N * C), TLN)
  y, p = _mm1_stats(xe0, K1Eopen.T, TLE)
  xer = _ntm_pad(y, K2Eopen.T, _mv(p, E * C), TLE)

  zeros_rows = jnp.zeros((N, CP), jnp.float32)
  ei3 = edge_i.reshape(NS, STEPS, CH)
  ej3 = edge_j.reshape(NS, STEPS, CH)

  for i in range(KE1.shape[0]):
    ke1a, ke1b = KE1[i][:, :C], KE1[i][:, C:]
    wi = (0.5 * ke1a + ke1b).T
    wj = (0.5 * ke1a - ke1b).T
    kn1a, kn1b = KN1[i][:, :C], KN1[i][:, C:]
    vi = (0.5 * kn1a + kn1b).T
    vj = (0.5 * kn1a - kn1b).T

    gi, gj = _sc_gather(xnr, ei3, ej3)
    y1, p1 = _mm2_stats(gi, gj, wi, wj, TLE)
    y2, p2 = _ntm_stats(y1, KE2[i].T, _mv(p1, E * C), TLE)
    xer = _axpy_norm(xer, y2, _mv(p2, E * C), TLE)

    si, sj = _sc_scatter(xer, ei3, ej3, zeros_rows)
    t1, p3 = _seg_mm2_stats(si, sj, vi, vj, TLN)
    xnr = _node_update(xnr, t1, KN2[i].T, _mv(p3, N * C), TLN)

  return _head(xnr, KNclose.T, lin1_w.T, lin1_b[None], lin2_w.T, lin2_b[None],
               TLN)


# speedup vs baseline: 3.4792x; 1.0420x over previous
import jax
import jax.numpy as jnp
from jax import lax
from jax.experimental import pallas as pl
from jax.experimental.pallas import tpu as pltpu
from jax.experimental.pallas import tpu_sc as plsc

N = 10000
E = 320000
C = 64
CP = 128
H = 0.1

NC = 2
NS = 16
CH = 80
PER_SW = E // NS
STEPS = PER_SW // CH
NPAIR = STEPS // 2
ROWS_W = 624
TAIL_W = N - NS * ROWS_W

TLE = 8000
TLN = 5000


def _sc_gather_body(xn_hbm, ei3_hbm, ej3_hbm, gi_hbm, gj_hbm,
                    idx_v, bufs, gsems, ssems):
  c = lax.axis_index("c")
  s = lax.axis_index("s")

  def side(idx3_hbm, out_hbm):
    pltpu.sync_copy(idx3_hbm.at[s], idx_v)
    base0 = s * PER_SW

    def run(kbase, width):
      ds_ = [pltpu.async_copy(xn_hbm.at[idx_v.at[kbase + u]], bufs[u],
                              gsems[u]) for u in range(width)]
      sts = []
      for u in range(width):
        ds_[u].wait()
        sts.append(pltpu.async_copy(
            bufs[u], out_hbm.at[pl.ds(base0 + (kbase + u) * CH, CH)],
            ssems[u]))
      for st in sts:
        st.wait()

    def quad(q, carry):
      run(4 * q, 4)
      return carry

    lax.fori_loop(0, STEPS // 4, quad, 0)
    run((STEPS // 4) * 4, STEPS % 4)

  @pl.when(c == 0)
  def _():
    side(ei3_hbm, gi_hbm)

  @pl.when(c == 1)
  def _():
    side(ej3_hbm, gj_hbm)


def _sc_gather(xn_rows, ei3, ej3):
  return pl.kernel(
      _sc_gather_body,
      out_type=(
          jax.ShapeDtypeStruct((E, CP), jnp.float32),
          jax.ShapeDtypeStruct((E, CP), jnp.float32),
      ),
      mesh=plsc.VectorSubcoreMesh(core_axis_name="c", subcore_axis_name="s"),
      scratch_types=[
          pltpu.VMEM((STEPS, CH), jnp.int32),
          [pltpu.VMEM((CH, CP), jnp.float32)] * 4,
          [pltpu.SemaphoreType.DMA] * 4,
          [pltpu.SemaphoreType.DMA] * 4,
      ],
  )(xn_rows, ei3, ej3)


def _sc_scatter_body(xe_hbm, ei3_hbm, ej3_hbm, zeros_hbm, si_hbm, sj_hbm,
                     idx_v, bufs, acc, lsems, asems):
  c = lax.axis_index("c")
  s = lax.axis_index("s")
  r0 = s * ROWS_W
  rt = NS * ROWS_W

  pltpu.sync_copy(zeros_hbm.at[pl.ds(r0, ROWS_W)], acc.at[pl.ds(r0, ROWS_W)])

  @pl.when(s == NS - 1)
  def _():
    pltpu.sync_copy(zeros_hbm.at[pl.ds(rt, TAIL_W)], acc.at[pl.ds(rt, TAIL_W)])

  plsc.subcore_barrier()

  def side(idx3_hbm, out_hbm):
    base0 = s * PER_SW

    def run(kbase, width, idxoff):
      ds_ = [pltpu.async_copy(xe_hbm.at[pl.ds(base0 + (kbase + u) * CH, CH)],
                              bufs[u], lsems[u]) for u in range(width)]
      adds = []
      for u in range(width):
        ds_[u].wait()
        adds.append(pltpu.async_copy(bufs[u], acc.at[idx_v.at[idxoff + u]],
                                     asems[u], add=True))
      for a in adds:
        a.wait()

    def octet(q, carry):
      k8 = 8 * q
      pltpu.sync_copy(idx3_hbm.at[s, pl.ds(k8, 8)], idx_v)
      run(k8, 4, 0)
      run(k8 + 4, 4, 4)
      return carry

    lax.fori_loop(0, STEPS // 8, octet, 0)
    kt = (STEPS // 8) * 8
    pltpu.sync_copy(idx3_hbm.at[s, pl.ds(kt, STEPS % 8)],
                    idx_v.at[pl.ds(0, STEPS % 8)])
    run(kt, STEPS % 8, 0)
    plsc.subcore_barrier()
    pltpu.sync_copy(acc.at[pl.ds(r0, ROWS_W)], out_hbm.at[pl.ds(r0, ROWS_W)])

    @pl.when(s == NS - 1)
    def _():
      pltpu.sync_copy(acc.at[pl.ds(rt, TAIL_W)], out_hbm.at[pl.ds(rt, TAIL_W)])

  @pl.when(c == 0)
  def _():
    side(ei3_hbm, si_hbm)

  @pl.when(c == 1)
  def _():
    side(ej3_hbm, sj_hbm)


def _sc_scatter(xe_rows, ei3, ej3, zeros_rows):
  return pl.kernel(
      _sc_scatter_body,
      out_type=(
          jax.ShapeDtypeStruct((N, CP), jnp.float32),
          jax.ShapeDtypeStruct((N, CP), jnp.float32),
      ),
      mesh=plsc.VectorSubcoreMesh(core_axis_name="c", subcore_axis_name="s"),
      scratch_types=[
          pltpu.VMEM((8, CH), jnp.int32),
          [pltpu.VMEM((CH, CP), jnp.float32)] * 4,
          pltpu.VMEM_SHARED((N, CP), jnp.float32),
          [pltpu.SemaphoreType.DMA] * 4,
          [pltpu.SemaphoreType.DMA] * 4,
      ],
  )(xe_rows, ei3, ej3, zeros_rows)


def _stats(y):
  s1 = jnp.sum(y)
  s2 = jnp.sum(y * y)
  col = lax.broadcasted_iota(jnp.int32, (1, 8, 128), 2)
  row = lax.broadcasted_iota(jnp.int32, (1, 8, 128), 1)
  vals = jnp.where(col == 0, s1, jnp.where(col == 1, s2, 0.0))
  return jnp.where(row == 0, vals, 0.0)


def _pad(y):
  return jnp.concatenate([y, jnp.zeros_like(y)], axis=1)


def _mm1_stats_body(x_ref, w_ref, y_ref, p_ref):
  y = jnp.dot(x_ref[...], w_ref[...], preferred_element_type=jnp.float32)
  y_ref[...] = y
  p_ref[...] = _stats(y)


def _mm1_stats(x, w, tl):
  rows, cin = x.shape
  cout = w.shape[1]
  g = rows // tl
  return pl.pallas_call(
      _mm1_stats_body,
      grid=(g,),
      in_specs=[
          pl.BlockSpec((tl, cin), lambda i: (i, 0)),
          pl.BlockSpec((cin, cout), lambda i: (0, 0)),
      ],
      out_specs=[
          pl.BlockSpec((tl, cout), lambda i: (i, 0)),
          pl.BlockSpec((1, 8, 128), lambda i: (i, 0, 0)),
      ],
      out_shape=[
          jax.ShapeDtypeStruct((rows, cout), jnp.float32),
          jax.ShapeDtypeStruct((g, 8, 128), jnp.float32),
      ],
  )(x, w)


def _mm2_stats_body(x1_ref, x2_ref, w1_ref, w2_ref, y_ref, p_ref):
  y = jnp.dot(x1_ref[:, :C], w1_ref[...], preferred_element_type=jnp.float32)
  y += jnp.dot(x2_ref[:, :C], w2_ref[...], preferred_element_type=jnp.float32)
  y_ref[...] = y
  p_ref[...] = _stats(y)


def _mm2_stats(x1, x2, w1, w2, tl):
  rows = x1.shape[0]
  g = rows // tl
  return pl.pallas_call(
      _mm2_stats_body,
      grid=(g,),
      in_specs=[
          pl.BlockSpec((tl, CP), lambda i: (i, 0)),
          pl.BlockSpec((tl, CP), lambda i: (i, 0)),
          pl.BlockSpec((C, C), lambda i: (0, 0)),
          pl.BlockSpec((C, C), lambda i: (0, 0)),
      ],
      out_specs=[
          pl.BlockSpec((tl, C), lambda i: (i, 0)),
          pl.BlockSpec((1, 8, 128), lambda i: (i, 0, 0)),
      ],
      out_shape=[
          jax.ShapeDtypeStruct((rows, C), jnp.float32),
          jax.ShapeDtypeStruct((g, 8, 128), jnp.float32),
      ],
  )(x1, x2, w1, w2)


def _seg_mm2_stats_body(si_ref, sj_ref, w1_ref, w2_ref, y_ref, p_ref):
  y = jnp.dot(si_ref[:, :C], w1_ref[...], preferred_element_type=jnp.float32)
  y += jnp.dot(sj_ref[:, :C], w2_ref[...], preferred_element_type=jnp.float32)
  y_ref[...] = y
  p_ref[...] = _stats(y)


def _seg_mm2_stats(si, sj, w1, w2, tl):
  rows = si.shape[0]
  g = rows // tl
  return pl.pallas_call(
      _seg_mm2_stats_body,
      grid=(g,),
      in_specs=[
          pl.BlockSpec((tl, CP), lambda i: (i, 0)),
          pl.BlockSpec((tl, CP), lambda i: (i, 0)),
          pl.BlockSpec((C, C), lambda i: (0, 0)),
          pl.BlockSpec((C, C), lambda i: (0, 0)),
      ],
      out_specs=[
          pl.BlockSpec((tl, C), lambda i: (i, 0)),
          pl.BlockSpec((1, 8, 128), lambda i: (i, 0, 0)),
      ],
      out_shape=[
          jax.ShapeDtypeStruct((rows, C), jnp.float32),
          jax.ShapeDtypeStruct((g, 8, 128), jnp.float32),
      ],
  )(si, sj, w1, w2)


def _ntm_body(y1_ref, w_ref, mv_ref, out_ref):
  m = mv_ref[0, 0]
  r = mv_ref[0, 1]
  z = jnp.tanh((y1_ref[...] - m) * r)
  out_ref[...] = jnp.dot(z, w_ref[...], preferred_element_type=jnp.float32)


def _ntm(y1, w, mv, tl):
  rows, cin = y1.shape
  cout = w.shape[1]
  g = rows // tl
  return pl.pallas_call(
      _ntm_body,
      grid=(g,),
      in_specs=[
          pl.BlockSpec((tl, cin), lambda i: (i, 0)),
          pl.BlockSpec((cin, cout), lambda i: (0, 0)),
          pl.BlockSpec((8, 128), lambda i: (0, 0)),
      ],
      out_specs=pl.BlockSpec((tl, cout), lambda i: (i, 0)),
      out_shape=jax.ShapeDtypeStruct((rows, cout), jnp.float32),
  )(y1, w, mv)


def _ntm_pad_body(y1_ref, w_ref, mv_ref, out_ref):
  m = mv_ref[0, 0]
  r = mv_ref[0, 1]
  z = jnp.tanh((y1_ref[...] - m) * r)
  out_ref[...] = _pad(
      jnp.dot(z, w_ref[...], preferred_element_type=jnp.float32))


def _ntm_pad(y1, w, mv, tl):
  rows, cin = y1.shape
  g = rows // tl
  return pl.pallas_call(
      _ntm_pad_body,
      grid=(g,),
      in_specs=[
          pl.BlockSpec((tl, cin), lambda i: (i, 0)),
          pl.BlockSpec((cin, C), lambda i: (0, 0)),
          pl.BlockSpec((8, 128), lambda i: (0, 0)),
      ],
      out_specs=pl.BlockSpec((tl, CP), lambda i: (i, 0)),
      out_shape=jax.ShapeDtypeStruct((rows, CP), jnp.float32),
  )(y1, w, mv)


def _ntm_stats_body(y1_ref, w_ref, mv_ref, out_ref, p_ref):
  m = mv_ref[0, 0]
  r = mv_ref[0, 1]
  z = jnp.tanh((y1_ref[...] - m) * r)
  y = jnp.dot(z, w_ref[...], preferred_element_type=jnp.float32)
  out_ref[...] = y
  p_ref[...] = _stats(y)


def _ntm_stats(y1, w, mv, tl):
  rows = y1.shape[0]
  g = rows // tl
  return pl.pallas_call(
      _ntm_stats_body,
      grid=(g,),
      in_specs=[
          pl.BlockSpec((tl, C), lambda i: (i, 0)),
          pl.BlockSpec((C, C), lambda i: (0, 0)),
          pl.BlockSpec((8, 128), lambda i: (0, 0)),
      ],
      out_specs=[
          pl.BlockSpec((tl, C), lambda i: (i, 0)),
          pl.BlockSpec((1, 8, 128), lambda i: (i, 0, 0)),
      ],
      out_shape=[
          jax.ShapeDtypeStruct((rows, C), jnp.float32),
          jax.ShapeDtypeStruct((g, 8, 128), jnp.float32),
      ],
  )(y1, w, mv)


def _axpy_norm_body(xe_ref, y2_ref, mv_ref, out_ref):
  m = mv_ref[0, 0]
  r = mv_ref[0, 1]
  upd = xe_ref[:, :C] + H * ((y2_ref[...] - m) * r)
  out_ref[...] = _pad(upd)


def _axpy_norm(xe, y2, mv, tl):
  rows = xe.shape[0]
  g = rows // tl
  return pl.pallas_call(
      _axpy_norm_body,
      grid=(g,),
      in_specs=[
          pl.BlockSpec((tl, CP), lambda i: (i, 0)),
          pl.BlockSpec((tl, C), lambda i: (i, 0)),
          pl.BlockSpec((8, 128), lambda i: (0, 0)),
      ],
      out_specs=pl.BlockSpec((tl, CP), lambda i: (i, 0)),
      out_shape=jax.ShapeDtypeStruct((rows, CP), jnp.float32),
  )(xe, y2, mv)


def _node_update_body(xn_ref, t1_ref, w_ref, mv_ref, out_ref):
  m = mv_ref[0, 0]
  r = mv_ref[0, 1]
  z = jnp.tanh((t1_ref[...] - m) * r)
  upd = xn_ref[:, :C] + H * jnp.dot(
      z, w_ref[...], preferred_element_type=jnp.float32)
  out_ref[...] = _pad(upd)


def _node_update(xn, t1, w, mv, tl):
  rows = xn.shape[0]
  g = rows // tl
  return pl.pallas_call(
      _node_update_body,
      grid=(g,),
      in_specs=[
          pl.BlockSpec((tl, CP), lambda i: (i, 0)),
          pl.BlockSpec((tl, C), lambda i: (i, 0)),
          pl.BlockSpec((C, C), lambda i: (0, 0)),
          pl.BlockSpec((8, 128), lambda i: (0, 0)),
      ],
      out_specs=pl.BlockSpec((tl, CP), lambda i: (i, 0)),
      out_shape=jax.ShapeDtypeStruct((rows, CP), jnp.float32),
  )(xn, t1, w, mv)


def _head_body(x_ref, kc_ref, w1_ref, b1_ref, w2_ref, b2_ref, o_ref):
  cvals = jnp.dot(x_ref[:, :C], kc_ref[...], preferred_element_type=jnp.float32)
  h = jnp.dot(cvals, w1_ref[...], preferred_element_type=jnp.float32)
  h += b1_ref[...]
  h = jnp.where(h > 0, h, jnp.exp(h) - 1.0)
  o = jnp.dot(h, w2_ref[...], preferred_element_type=jnp.float32)
  o_ref[...] = o + b2_ref[...]


def _head(x, kc, w1, b1, w2, b2, tl):
  rows = x.shape[0]
  g = rows // tl
  return pl.pallas_call(
      _head_body,
      grid=(g,),
      in_specs=[
          pl.BlockSpec((tl, CP), lambda i: (i, 0)),
          pl.BlockSpec((C, C), lambda i: (0, 0)),
          pl.BlockSpec((C, 256), lambda i: (0, 0)),
          pl.BlockSpec((1, 256), lambda i: (0, 0)),
          pl.BlockSpec((256, 1024), lambda i: (0, 0)),
          pl.BlockSpec((1, 1024), lambda i: (0, 0)),
      ],
      out_specs=pl.BlockSpec((tl, 1024), lambda i: (i, 0)),
      out_shape=jax.ShapeDtypeStruct((rows, 1024), jnp.float32),
  )(x, kc, w1, b1, w2, b2)


def _mv(p, count):
  s1 = jnp.sum(p[:, 0, 0])
  s2 = jnp.sum(p[:, 0, 1])
  m = s1 / count
  v = s2 / count - m * m
  r = lax.rsqrt(v + 1e-5)
  return jnp.zeros((8, 128), jnp.float32).at[0, 0].set(m).at[0, 1].set(r)


def kernel(xn, xe, edge_i, edge_j, K1Nopen, K2Nopen, K1Eopen, K2Eopen,
           KNclose, KE1, KE2, KN1, KN2, lin1_w, lin1_b, lin2_w, lin2_b):
  xn0 = xn[0].T
  xe0 = xe[0].T

  y, p = _mm1_stats(xn0, K1Nopen.T, TLN)
  xnr = _ntm_pad(y, K2Nopen.T, _mv(p, N * C), TLN)
  y, p = _mm1_stats(xe0, K1Eopen.T, TLE)
  xer = _ntm_pad(y, K2Eopen.T, _mv(p, E * C), TLE)

  zeros_rows = jnp.zeros((N, CP), jnp.float32)
  ei3 = edge_i.reshape(NS, STEPS, CH)
  ej3 = edge_j.reshape(NS, STEPS, CH)

  for i in range(KE1.shape[0]):
    ke1a, ke1b = KE1[i][:, :C], KE1[i][:, C:]
    wi = (0.5 * ke1a + ke1b).T
    wj = (0.5 * ke1a - ke1b).T
    kn1a, kn1b = KN1[i][:, :C], KN1[i][:, C:]
    vi = (0.5 * kn1a + kn1b).T
    vj = (0.5 * kn1a - kn1b).T

    gi, gj = _sc_gather(xnr, ei3, ej3)
    y1, p1 = _mm2_stats(gi, gj, wi, wj, TLE)
    y2, p2 = _ntm_stats(y1, KE2[i].T, _mv(p1, E * C), TLE)
    xer = _axpy_norm(xer, y2, _mv(p2, E * C), TLE)

    si, sj = _sc_scatter(xer, ei3, ej3, zeros_rows)
    t1, p3 = _seg_mm2_stats(si, sj, vi, vj, TLN)
    xnr = _node_update(xnr, t1, KN2[i].T, _mv(p3, N * C), TLN)

  return _head(xnr, KNclose.T, lin1_w.T, lin1_b[None], lin2_w.T, lin2_b[None],
               TLN)
